# Initial kernel scaffold; baseline (speedup 1.0000x reference)
import jax
import jax.numpy as jnp
from jax.experimental import pallas as pl


def kernel(x, edge_index, edge_weight, W, b):
    raise NotImplementedError("write your pallas kernel here")


# speedup vs baseline: 4.2218x; 4.2218x over previous
import functools

import jax
import jax.numpy as jnp
from jax import lax
from jax.experimental import pallas as pl
from jax.experimental.pallas import tpu as pltpu
from jax.experimental.pallas import tpu_sc as plsc

_N = 10000
_D = 128
_E = 320000
_NC = 2
_NS = 16
_NW = _NC * _NS
_EPW = _E // _NW
_C = 80
_NCH = _EPW // _C
_RPT = _N // _NS
_ZR = 125
_OPT = 624
_OTAIL = _N - _NS * _OPT


def _spmm_body(x_hbm, src_hbm, dst_hbm, ew_hbm, y_hbm,
               yacc, srcv, dstv, eww, rows, zbuf, sem):
    cid = lax.axis_index("c")
    sid = lax.axis_index("s")
    row0 = sid * _RPT

    def _zrow(r, carry):
        for j in range(_D // 16):
            zbuf[r, pl.ds(j * 16, 16)] = jnp.zeros((16,), jnp.float32)
        return carry

    lax.fori_loop(0, _ZR, _zrow, 0)
    for cz in range(_RPT // _ZR):
        pltpu.sync_copy(zbuf, yacc.at[pl.ds(row0 + cz * _ZR, _ZR)])
    plsc.subcore_barrier()

    ebase = (cid * _NS + sid) * _EPW

    def _chunk(k, carry):
        base = ebase + k * _C
        pltpu.sync_copy(src_hbm.at[pl.ds(base, _C)], srcv)
        pltpu.sync_copy(dst_hbm.at[pl.ds(base, _C)], dstv)
        pltpu.sync_copy(ew_hbm.at[pl.ds(base, _C)], eww)
        pltpu.async_copy(x_hbm.at[srcv], rows, sem).wait()

        def _scale(g, c2):
            wv = eww[pl.ds(g * 16, 16)]
            for l in range(16):
                w = jnp.full((16,), wv[l], jnp.float32)
                r = g * 16 + l
                for j in range(_D // 16):
                    sl = pl.ds(j * 16, 16)
                    rows[r, sl] = rows[r, sl] * w
            return c2

        lax.fori_loop(0, _C // 16, _scale, 0)
        pltpu.sync_copy(rows, yacc.at[dstv], add=True)
        return carry

    lax.fori_loop(0, _NCH, _chunk, 0)
    plsc.subcore_barrier()

    out0 = sid * _OPT
    pltpu.sync_copy(yacc.at[pl.ds(out0, _OPT)],
                    y_hbm.at[cid, pl.ds(out0, _OPT)])

    @pl.when(sid == _NS - 1)
    def _tail():
        pltpu.sync_copy(yacc.at[pl.ds(_NS * _OPT, _OTAIL)],
                        y_hbm.at[cid, pl.ds(_NS * _OPT, _OTAIL)])


_spmm = functools.partial(
    pl.kernel,
    out_type=jax.ShapeDtypeStruct((_NC, _N, _D), jnp.float32),
    mesh=plsc.VectorSubcoreMesh(core_axis_name="c", subcore_axis_name="s"),
    scratch_types=[
        pltpu.VMEM_SHARED((_N, _D), jnp.float32),
        pltpu.VMEM((_C,), jnp.int32),
        pltpu.VMEM((_C,), jnp.int32),
        pltpu.VMEM((_C,), jnp.float32),
        pltpu.VMEM((_C, _D), jnp.float32),
        pltpu.VMEM((_ZR, _D), jnp.float32),
        pltpu.SemaphoreType.DMA,
    ],
)(_spmm_body)


_BLK = 2000


def _mm_body(x_ref, y_ref, w_ref, b_ref, o_ref):
    kk = w_ref.shape[0]
    w1 = w_ref[0]
    w2 = jnp.zeros((_D, _D), jnp.float32)
    for k in range(1, kk):
        w1 = w1 + w_ref[k]
        w2 = w2 + float(k) * w_ref[k]
    ys = y_ref[0] + y_ref[1]
    o_ref[...] = (jnp.dot(x_ref[...], w1, preferred_element_type=jnp.float32)
                  - jnp.dot(ys, w2, preferred_element_type=jnp.float32)
                  + b_ref[...])


def _fused_mm(x, y2, W, b2):
    return pl.pallas_call(
        _mm_body,
        grid=(_N // _BLK,),
        in_specs=[
            pl.BlockSpec((_BLK, _D), lambda i: (i, 0)),
            pl.BlockSpec((_NC, _BLK, _D), lambda i: (0, i, 0)),
            pl.BlockSpec((W.shape[0], _D, _D), lambda i: (0, 0, 0)),
            pl.BlockSpec((1, _D), lambda i: (0, 0)),
        ],
        out_specs=pl.BlockSpec((_BLK, _D), lambda i: (i, 0)),
        out_shape=jax.ShapeDtypeStruct((_N, _D), jnp.float32),
    )(x, y2, W, b2)


def kernel(x, edge_index, edge_weight, W, b):
    y2 = _spmm(x, edge_index[0], edge_index[1], edge_weight)
    return _fused_mm(x, y2, W, b.reshape(1, _D))


# speedup vs baseline: 8.6861x; 2.0575x over previous
import functools

import jax
import jax.numpy as jnp
from jax import lax
from jax.experimental import pallas as pl
from jax.experimental.pallas import tpu as pltpu
from jax.experimental.pallas import tpu_sc as plsc

_N = 10000
_D = 128
_E = 320000
_NC = 2
_NS = 16
_NW = _NC * _NS
_EPW = _E // _NW
_C = 80
_NCH = _EPW // _C
_RPT = _N // _NS
_OPT = 624
_OTAIL = _N - _NS * _OPT


def _spmm_body(x_hbm, src_hbm, dst_hbm, ew_hbm, y_hbm, yacc,
               srcv_a, dstv_a, eww_a, srcv_b, dstv_b, eww_b,
               rows_a, rows_b, sem_ea, sem_eb, sem_ga, sem_gb):
    cid = lax.axis_index("c")
    sid = lax.axis_index("s")
    row0 = sid * _RPT
    ebase = (cid * _NS + sid) * _EPW

    srcv = (srcv_a, srcv_b)
    dstv = (dstv_a, dstv_b)
    eww = (eww_a, eww_b)
    rows = (rows_a, rows_b)
    sem_e = (sem_ea, sem_eb)
    sem_g = (sem_ga, sem_gb)

    def _stage(k, s):
        base = ebase + k * _C
        pltpu.async_copy(src_hbm.at[pl.ds(base, _C)], srcv[s], sem_e[s])
        pltpu.async_copy(dst_hbm.at[pl.ds(base, _C)], dstv[s], sem_e[s])
        pltpu.async_copy(ew_hbm.at[pl.ds(base, _C)], eww[s], sem_e[s])

    def _wait_stage(k, s):
        base = ebase + k * _C
        pltpu.make_async_copy(src_hbm.at[pl.ds(base, _C)], srcv[s], sem_e[s]).wait()
        pltpu.make_async_copy(dst_hbm.at[pl.ds(base, _C)], dstv[s], sem_e[s]).wait()
        pltpu.make_async_copy(ew_hbm.at[pl.ds(base, _C)], eww[s], sem_e[s]).wait()

    def _zrow(r, carry):
        for j in range(_D // 16):
            rows_a[r, pl.ds(j * 16, 16)] = jnp.zeros((16,), jnp.float32)
        return carry

    lax.fori_loop(0, _C, _zrow, 0)
    for cz in range(_RPT // _C):
        pltpu.sync_copy(rows_a, yacc.at[pl.ds(row0 + cz * _C, _C)])
    _zt = _RPT - (_RPT // _C) * _C
    if _zt:
        pltpu.sync_copy(rows_a.at[pl.ds(0, _zt)],
                        yacc.at[pl.ds(row0 + (_RPT // _C) * _C, _zt)])
    plsc.subcore_barrier()

    _stage(0, 0)
    _wait_stage(0, 0)
    pltpu.async_copy(x_hbm.at[srcv[0]], rows[0], sem_g[0])
    _stage(1, 1)

    def _chunk(k, cur, oth):
        @pl.when(k + 1 < _NCH)
        def _next_gather():
            _wait_stage(k + 1, oth)
            pltpu.async_copy(x_hbm.at[srcv[oth]], rows[oth], sem_g[oth])

        pltpu.make_async_copy(x_hbm.at[srcv[cur]], rows[cur], sem_g[cur]).wait()

        def _scale(g, c2):
            wv = eww[cur][pl.ds(g * 16, 16)]
            for l in range(16):
                w = jnp.full((16,), wv[l], jnp.float32)
                r = g * 16 + l
                for j in range(_D // 16):
                    sl = pl.ds(j * 16, 16)
                    rows[cur][r, sl] = rows[cur][r, sl] * w
            return c2

        lax.fori_loop(0, _C // 16, _scale, 0)
        pltpu.sync_copy(rows[cur], yacc.at[dstv[cur]], add=True)

        @pl.when(k + 2 < _NCH)
        def _next_stage():
            _stage(k + 2, cur)

    def _chunk2(k2, carry):
        k = k2 * 2
        _chunk(k, 0, 1)
        _chunk(k + 1, 1, 0)
        return carry

    lax.fori_loop(0, _NCH // 2, _chunk2, 0)
    _chunk(_NCH - 1, 0, 1)
    plsc.subcore_barrier()

    out0 = sid * _OPT
    pltpu.sync_copy(yacc.at[pl.ds(out0, _OPT)],
                    y_hbm.at[cid, pl.ds(out0, _OPT)])

    @pl.when(sid == _NS - 1)
    def _tail():
        pltpu.sync_copy(yacc.at[pl.ds(_NS * _OPT, _OTAIL)],
                        y_hbm.at[cid, pl.ds(_NS * _OPT, _OTAIL)])


_spmm = functools.partial(
    pl.kernel,
    out_type=jax.ShapeDtypeStruct((_NC, _N, _D), jnp.float32),
    mesh=plsc.VectorSubcoreMesh(core_axis_name="c", subcore_axis_name="s"),
    scratch_types=[
        pltpu.VMEM_SHARED((_N, _D), jnp.float32),
        pltpu.VMEM((_C,), jnp.int32),
        pltpu.VMEM((_C,), jnp.int32),
        pltpu.VMEM((_C,), jnp.float32),
        pltpu.VMEM((_C,), jnp.int32),
        pltpu.VMEM((_C,), jnp.int32),
        pltpu.VMEM((_C,), jnp.float32),
        pltpu.VMEM((_C, _D), jnp.float32),
        pltpu.VMEM((_C, _D), jnp.float32),
        pltpu.SemaphoreType.DMA,
        pltpu.SemaphoreType.DMA,
        pltpu.SemaphoreType.DMA,
        pltpu.SemaphoreType.DMA,
    ],
)(_spmm_body)


_BLK = 2000


def _mm_body(x_ref, y_ref, w_ref, b_ref, o_ref):
    kk = w_ref.shape[0]
    w1 = w_ref[0]
    w2 = jnp.zeros((_D, _D), jnp.float32)
    for k in range(1, kk):
        w1 = w1 + w_ref[k]
        w2 = w2 + float(k) * w_ref[k]
    ys = y_ref[0] + y_ref[1]
    o_ref[...] = (jnp.dot(x_ref[...], w1, preferred_element_type=jnp.float32)
                  - jnp.dot(ys, w2, preferred_element_type=jnp.float32)
                  + b_ref[...])


def _fused_mm(x, y2, W, b2):
    return pl.pallas_call(
        _mm_body,
        grid=(_N // _BLK,),
        in_specs=[
            pl.BlockSpec((_BLK, _D), lambda i: (i, 0)),
            pl.BlockSpec((_NC, _BLK, _D), lambda i: (0, i, 0)),
            pl.BlockSpec((W.shape[0], _D, _D), lambda i: (0, 0, 0)),
            pl.BlockSpec((1, _D), lambda i: (0, 0)),
        ],
        out_specs=pl.BlockSpec((_BLK, _D), lambda i: (i, 0)),
        out_shape=jax.ShapeDtypeStruct((_N, _D), jnp.float32),
    )(x, y2, W, b2)


def kernel(x, edge_index, edge_weight, W, b):
    y2 = _spmm(x, edge_index[0], edge_index[1], edge_weight)
    return _fused_mm(x, y2, W, b.reshape(1, _D))
